# baseline (speedup 1.0000x reference)
import functools

import jax
import jax.numpy as jnp
from jax import lax
from jax.experimental import pallas as pl
from jax.experimental.pallas import tpu as pltpu
from jax.experimental.pallas import tpu_sc as plsc

VOCAB = 49408
HIDDEN = 768
MAX_POS = 77
BATCH = 1024
SEQ = 77

NC = 2
NS = 16
NW = NC * NS

BPW = BATCH // NW
PG = 8
NPG = 10
GB = 4
NGB = BPW // GB
NCHUNKS = NPG * NGB
ROWS = GB * PG
TAIL = SEQ - (NPG - 1) * PG
LANES = 16
NVEC = HIDDEN // LANES
NBUF = 2


def _body(table_hbm, idx_hbm, pos_hbm, out_hbm, idx_v, pos_v, buf, gsem, ssem):
    wid = lax.axis_index("s") * NC + lax.axis_index("c")
    b00 = wid * BPW

    pltpu.sync_copy(idx_hbm.at[wid], idx_v)
    pltpu.sync_copy(pos_hbm, pos_v)

    def gather_start(c):
        m = lax.rem(c, NBUF)
        pltpu.async_copy(table_hbm.at[idx_v.at[c]], buf.at[m], gsem)

    def gather_wait(c):
        m = lax.rem(c, NBUF)
        pltpu.make_async_copy(table_hbm.at[idx_v.at[c]], buf.at[m], gsem).wait()

    def _scatters(c, fn):
        m = lax.rem(c, NBUF)
        r = lax.div(c, NGB)
        b0 = b00 + lax.rem(c, NGB) * GB

        @pl.when(r < NPG - 1)
        def _():
            for bi in range(GB):
                fn(buf.at[m, pl.ds(bi * PG, PG)],
                   out_hbm.at[b0 + bi, pl.ds(r * PG, PG)])

        @pl.when(r == NPG - 1)
        def _():
            for bi in range(GB):
                fn(buf.at[m, pl.ds(bi * PG, TAIL)],
                   out_hbm.at[b0 + bi, pl.ds((NPG - 1) * PG, TAIL)])

    def scatter_start(c):
        _scatters(c, lambda src, dst: pltpu.async_copy(src, dst, ssem))

    def scatter_wait(c):
        _scatters(c, lambda src, dst: pltpu.make_async_copy(src, dst, ssem).wait())

    gather_start(0)

    def chunk_body(c, _):
        @pl.when(c >= 1)
        def _():
            scatter_wait(c - 1)

        @pl.when(c + 1 < NCHUNKS)
        def _():
            gather_start(c + 1)

        gather_wait(c)
        m = lax.rem(c, NBUF)
        p0 = lax.div(c, NGB) * PG

        def col_body(j, _):
            sl = pl.ds(j * LANES, LANES)
            for si in range(PG):
                pv = pos_v[p0 + si, sl]
                for bi in range(GB):
                    buf[m, bi * PG + si, sl] += pv
            return 0

        lax.fori_loop(0, NVEC, col_body, 0)

        scatter_start(c)
        return 0

    lax.fori_loop(0, NCHUNKS, chunk_body, 0)
    scatter_wait(NCHUNKS - 1)


_sc_call = functools.partial(
    pl.kernel,
    out_type=jax.ShapeDtypeStruct((BATCH, SEQ, HIDDEN), jnp.float32),
    mesh=plsc.VectorSubcoreMesh(
        core_axis_name="c", subcore_axis_name="s", num_cores=NC, num_subcores=NS
    ),
    scratch_types=[
        pltpu.VMEM((NCHUNKS, ROWS), jnp.int32),
        pltpu.VMEM((NPG * PG, HIDDEN), jnp.float32),
        pltpu.VMEM((NBUF, ROWS, HIDDEN), jnp.float32),
        pltpu.SemaphoreType.DMA,
        pltpu.SemaphoreType.DMA,
    ],
)(_body)


@jax.jit
def kernel(input_ids, token_table, pos_table):
    ids = input_ids.astype(jnp.int32)
    ids = jnp.pad(ids, ((0, 0), (0, NPG * PG - SEQ)), mode="edge")
    ids = ids.reshape(NW, NGB, GB, NPG, PG).transpose(0, 3, 1, 2, 4)
    ids = ids.reshape(NW, NCHUNKS, ROWS)
    pos = jnp.pad(pos_table, ((0, NPG * PG - SEQ), (0, 0)))
    return _sc_call(token_table, ids, pos)
